# baseline (speedup 1.0000x reference)
import jax
import jax.numpy as jnp
from jax.experimental import pallas as pl
from jax.experimental.pallas import tpu as pltpu

_NBUF = 4


def _pipeline_body(x_hbm, o_hbm, *scratch):
    in_bufs = scratch[0:_NBUF]
    out_bufs = scratch[_NBUF:2 * _NBUF]
    in_sems = scratch[2 * _NBUF:3 * _NBUF]
    out_sems = scratch[3 * _NBUF:4 * _NBUF]
    b = x_hbm.shape[0]

    def in_copy(i, slot):
        return pltpu.make_async_copy(x_hbm.at[i], in_bufs[slot], in_sems[slot])

    def out_copy(i, slot):
        return pltpu.make_async_copy(out_bufs[slot], o_hbm.at[i], out_sems[slot])

    for s in range(_NBUF):
        in_copy(s, s).start()
    for i in range(b):
        slot = i % _NBUF
        in_copy(i, slot).wait()
        if i >= _NBUF:
            out_copy(i - _NBUF, slot).wait()
        out_bufs[slot][...] = in_bufs[slot][...].T
        out_copy(i, slot).start()
        nxt = i + _NBUF
        if nxt < b:
            in_copy(nxt, slot).start()
    for i in range(b - _NBUF, b):
        out_copy(i, i % _NBUF).wait()


def kernel(input):
    b, e, h, w = input.shape
    hw = h * w
    x = input.reshape(b, e, hw)
    out = pl.pallas_call(
        _pipeline_body,
        in_specs=[pl.BlockSpec(memory_space=pltpu.MemorySpace.HBM)],
        out_specs=pl.BlockSpec(memory_space=pltpu.MemorySpace.HBM),
        out_shape=jax.ShapeDtypeStruct((b, hw, e), x.dtype),
        scratch_shapes=(
            [pltpu.VMEM((e, hw), x.dtype) for _ in range(_NBUF)]
            + [pltpu.VMEM((hw, e), x.dtype) for _ in range(_NBUF)]
            + [pltpu.SemaphoreType.DMA for _ in range(_NBUF)]
            + [pltpu.SemaphoreType.DMA for _ in range(_NBUF)]
        ),
    )(x)
    length = jnp.full((b,), True, dtype=bool)
    return (out, length)
